# Initial kernel scaffold; baseline (speedup 1.0000x reference)
import jax
import jax.numpy as jnp
from jax.experimental import pallas as pl


def kernel(inputs, src0, dst0, src1, dst1, W_self0, W_neigh0, b0, W_self1, W_neigh1, b1, W_prompt, W_pp):
    raise NotImplementedError("write your pallas kernel here")


# speedup vs baseline: 5.0173x; 5.0173x over previous
import jax
import jax.numpy as jnp
from jax import lax
from jax.experimental import pallas as pl
from jax.experimental.pallas import tpu as pltpu
from jax.experimental.pallas import tpu_sc as plsc

N0, N1, N2 = 50000, 16384, 4096
E0, E1 = 262144, 65536
D = 256
DC = 64
NCHUNK = D // DC
NCORE, NSUB = 2, 16
NW = NCORE * NSUB
F32 = jnp.float32
HI = lax.Precision.HIGHEST


def _fill(ref, rows, cols, val):
  v = jnp.full((16,), val, F32)

  def body(i, _):
    for k in range(cols // 16):
      ref[i, pl.ds(k * 16, 16)] = v
    return 0

  lax.fori_loop(0, rows, body, 0)


def _sc_l0_body(table4, src2d, dst2d, agg_part, deg_part,
                acc_sh, deg_sh, srcbuf, dstbuf, idxbuf, rows, ones, zdeg,
                sem):
  c = lax.axis_index("c")
  s = lax.axis_index("s")
  wid = c * NSUB + s

  _fill(ones, 128, 16, 1.0)
  _fill(zdeg, 128, 16, 0.0)

  def zacc(r, _):
    pltpu.sync_copy(rows, acc_sh.at[pl.ds(s * 1024 + r * 128, 128)])
    return 0

  def zdg(r, _):
    pltpu.sync_copy(zdeg, deg_sh.at[pl.ds(s * 1024 + r * 128, 128)])
    return 0

  pltpu.sync_copy(src2d.at[pl.ds(wid * 64, 64)], srcbuf)
  pltpu.sync_copy(dst2d.at[pl.ds(wid * 64, 64)], dstbuf)

  for cc in range(NCHUNK):
    _fill(rows, 128, DC, 0.0)
    lax.fori_loop(0, 8, zacc, 0)
    if cc == 0:
      lax.fori_loop(0, 8, zdg, 0)

    def mkidx(j, _):
      for k in range(8):
        v = srcbuf[j, pl.ds(k * 16, 16)]
        idxbuf[j, pl.ds(k * 16, 16)] = v * 4 + cc
      return 0

    lax.fori_loop(0, 64, mkidx, 0)
    plsc.subcore_barrier()

    def stream(j, _):
      pltpu.async_copy(table4.at[idxbuf.at[j]], rows, sem).wait()
      pltpu.sync_copy(rows, acc_sh.at[dstbuf.at[j]], add=True)
      return 0

    lax.fori_loop(0, 64, stream, 0)

    if cc == 0:
      def dstream(j, _):
        pltpu.sync_copy(ones, deg_sh.at[dstbuf.at[j]], add=True)
        return 0

      lax.fori_loop(0, 64, dstream, 0)

    plsc.subcore_barrier()
    pltpu.sync_copy(acc_sh.at[pl.ds(s * 1024, 1024)],
                    agg_part.at[c, cc, pl.ds(s * 1024, 1024)])
    if cc == 0:
      pltpu.sync_copy(deg_sh.at[pl.ds(s * 1024, 1024)],
                      deg_part.at[c, pl.ds(s * 1024, 1024)])


_sc_l0 = pl.kernel(
    _sc_l0_body,
    out_type=(
        jax.ShapeDtypeStruct((NCORE, NCHUNK, N1, DC), F32),
        jax.ShapeDtypeStruct((NCORE, N1, 16), F32),
    ),
    mesh=plsc.VectorSubcoreMesh(core_axis_name="c", subcore_axis_name="s"),
    compiler_params=pltpu.CompilerParams(use_tc_tiling_on_sc=False),
    scratch_types=[
        pltpu.VMEM_SHARED((N1, DC), F32),
        pltpu.VMEM_SHARED((N1, 16), F32),
        pltpu.VMEM((64, 128), jnp.int32),
        pltpu.VMEM((64, 128), jnp.int32),
        pltpu.VMEM((64, 128), jnp.int32),
        pltpu.VMEM((128, DC), F32),
        pltpu.VMEM((128, 16), F32),
        pltpu.VMEM((128, 16), F32),
        pltpu.SemaphoreType.DMA,
    ],
)


def _sc_l1_body(h1, src2d, dst2d, agg_part, deg_part,
                acc_sh, deg_sh, srcbuf, dstbuf, rows, ones, zdeg, sem):
  c = lax.axis_index("c")
  s = lax.axis_index("s")
  wid = c * NSUB + s

  _fill(rows, 128, D, 0.0)
  _fill(zdeg, 128, 16, 0.0)
  _fill(ones, 128, 16, 1.0)

  def zacc(r, _):
    pltpu.sync_copy(rows, acc_sh.at[pl.ds(s * 256 + r * 128, 128)])
    return 0

  def zdg(r, _):
    pltpu.sync_copy(zdeg, deg_sh.at[pl.ds(s * 256 + r * 128, 128)])
    return 0

  lax.fori_loop(0, 2, zacc, 0)
  lax.fori_loop(0, 2, zdg, 0)

  pltpu.sync_copy(src2d.at[pl.ds(wid * 16, 16)], srcbuf)
  pltpu.sync_copy(dst2d.at[pl.ds(wid * 16, 16)], dstbuf)
  plsc.subcore_barrier()

  def stream(j, _):
    pltpu.async_copy(h1.at[srcbuf.at[j]], rows, sem).wait()
    pltpu.sync_copy(rows, acc_sh.at[dstbuf.at[j]], add=True)
    pltpu.sync_copy(ones, deg_sh.at[dstbuf.at[j]], add=True)
    return 0

  lax.fori_loop(0, 16, stream, 0)

  plsc.subcore_barrier()
  pltpu.sync_copy(acc_sh.at[pl.ds(s * 256, 256)],
                  agg_part.at[c, pl.ds(s * 256, 256)])
  pltpu.sync_copy(deg_sh.at[pl.ds(s * 256, 256)],
                  deg_part.at[c, pl.ds(s * 256, 256)])


_sc_l1 = pl.kernel(
    _sc_l1_body,
    out_type=(
        jax.ShapeDtypeStruct((NCORE, N2, D), F32),
        jax.ShapeDtypeStruct((NCORE, N2, 16), F32),
    ),
    mesh=plsc.VectorSubcoreMesh(core_axis_name="c", subcore_axis_name="s"),
    compiler_params=pltpu.CompilerParams(use_tc_tiling_on_sc=False),
    scratch_types=[
        pltpu.VMEM_SHARED((N2, D), F32),
        pltpu.VMEM_SHARED((N2, 16), F32),
        pltpu.VMEM((16, 128), jnp.int32),
        pltpu.VMEM((16, 128), jnp.int32),
        pltpu.VMEM((128, D), F32),
        pltpu.VMEM((128, 16), F32),
        pltpu.VMEM((128, 16), F32),
        pltpu.SemaphoreType.DMA,
    ],
)


def _dense0_body(x_ref, ap_ref, dg_ref, ws_ref, wn_ref, b_ref, o_ref):
  ap = ap_ref[...]
  agg = ap[0] + ap[1]
  agg = jnp.concatenate([agg[0], agg[1], agg[2], agg[3]], axis=-1)
  deg = jnp.max(dg_ref[0] + dg_ref[1], axis=1)
  neigh = agg / jnp.clip(deg, 1.0, None)[:, None]
  h = (jnp.dot(x_ref[...], ws_ref[...], preferred_element_type=F32)
       + jnp.dot(neigh, wn_ref[...], preferred_element_type=F32)
       + b_ref[...])
  o_ref[...] = jnp.maximum(h, 0.0)


_dense0 = pl.pallas_call(
    _dense0_body,
    grid=(N1 // 512,),
    in_specs=[
        pl.BlockSpec((512, D), lambda i: (i, 0)),
        pl.BlockSpec((NCORE, NCHUNK, 512, DC), lambda i: (0, 0, i, 0)),
        pl.BlockSpec((NCORE, 512, 16), lambda i: (0, i, 0)),
        pl.BlockSpec((D, D), lambda i: (0, 0)),
        pl.BlockSpec((D, D), lambda i: (0, 0)),
        pl.BlockSpec((1, D), lambda i: (0, 0)),
    ],
    out_specs=pl.BlockSpec((512, D), lambda i: (i, 0)),
    out_shape=jax.ShapeDtypeStruct((N1, D), F32),
)


def _head_body(h_ref, ap_ref, dg_ref, ws_ref, wn_ref, b_ref, wp_ref, wpp_ref,
               o_ref):
  ap = ap_ref[...]
  deg = jnp.max(dg_ref[0] + dg_ref[1], axis=1)
  neigh = (ap[0] + ap[1]) / jnp.clip(deg, 1.0, None)[:, None]
  hd = h_ref[...]
  h2 = (jnp.dot(hd, ws_ref[...], preferred_element_type=F32)
        + jnp.dot(neigh, wn_ref[...], preferred_element_type=F32)
        + b_ref[...])
  h2 = jnp.maximum(h2, 0.0)
  hcat = jnp.concatenate([h2, jnp.maximum(hd, 0.0)], axis=1)
  logits = lax.dot_general(hcat, wp_ref[...], (((1,), (1,)), ((), ())))
  mx = jnp.max(logits, axis=1, keepdims=True)
  eq = (logits >= mx).astype(F32)
  tri = (lax.broadcasted_iota(jnp.int32, (8, 8), 0)
         <= lax.broadcasted_iota(jnp.int32, (8, 8), 1)).astype(F32)
  cs = jnp.dot(eq, tri)
  onehot = eq * (cs == 1.0).astype(F32)
  allout = lax.dot_general(hcat, wpp_ref[...], (((1,), (1,)), ((), ())))
  ccg = lax.broadcasted_iota(jnp.int32, (8, 128), 0)
  jjg = lax.broadcasted_iota(jnp.int32, (8, 128), 1)
  expand = (jjg // 16 == ccg).astype(F32)
  maskfull = jnp.dot(onehot, expand)
  jk = lax.broadcasted_iota(jnp.int32, (128, 16), 0)
  kk = lax.broadcasted_iota(jnp.int32, (128, 16), 1)
  fold = (jk % 16 == kk).astype(F32)
  o_ref[...] = jnp.dot(allout * maskfull, fold)


_head = pl.pallas_call(
    _head_body,
    grid=(N2 // 512,),
    in_specs=[
        pl.BlockSpec((512, D), lambda i: (i, 0)),
        pl.BlockSpec((NCORE, 512, D), lambda i: (0, i, 0)),
        pl.BlockSpec((NCORE, 512, 16), lambda i: (0, i, 0)),
        pl.BlockSpec((D, D), lambda i: (0, 0)),
        pl.BlockSpec((D, D), lambda i: (0, 0)),
        pl.BlockSpec((1, D), lambda i: (0, 0)),
        pl.BlockSpec((8, 2 * D), lambda i: (0, 0)),
        pl.BlockSpec((128, 2 * D), lambda i: (0, 0)),
    ],
    out_specs=pl.BlockSpec((512, 16), lambda i: (i, 0)),
    out_shape=jax.ShapeDtypeStruct((N2, 16), F32),
)


def kernel(inputs, src0, dst0, src1, dst1, W_self0, W_neigh0, b0, W_self1,
           W_neigh1, b1, W_prompt, W_pp):
  table4 = inputs.reshape(N0 * NCHUNK, DC)
  agg0, deg0 = _sc_l0(table4, src0.reshape(E0 // 128, 128),
                      dst0.reshape(E0 // 128, 128))
  h1 = _dense0(inputs[:N1], agg0, deg0, W_self0, W_neigh0, b0.reshape(1, D))
  agg1, deg1 = _sc_l1(h1, src1.reshape(E1 // 128, 128),
                      dst1.reshape(E1 // 128, 128))
  out = _head(h1, agg1, deg1, W_self1, W_neigh1, b1.reshape(1, D), W_prompt,
              W_pp.reshape(8 * 16, 2 * D))
  return out


# speedup vs baseline: 6.8117x; 1.3576x over previous
import jax
import jax.numpy as jnp
from jax import lax
from jax.experimental import pallas as pl
from jax.experimental.pallas import tpu as pltpu
from jax.experimental.pallas import tpu_sc as plsc

N0, N1, N2 = 50000, 16384, 4096
E0, E1 = 262144, 65536
D = 256
DC = 64
NCHUNK = D // DC
NCORE, NSUB = 2, 16
NW = NCORE * NSUB
F32 = jnp.float32
HI = lax.Precision.HIGHEST


def _fill(ref, rows, cols, val):
  v = jnp.full((16,), val, F32)

  def body(i, _):
    for k in range(cols // 16):
      ref[i, pl.ds(k * 16, 16)] = v
    return 0

  lax.fori_loop(0, rows, body, 0)


def _sc_l0_body(table4, src2d, dst2d, agg_part, deg_part,
                acc_sh, deg_sh, dstbuf, idxbuf, rows_a, rows_b, ones, zdeg,
                sem_a, sem_b):
  c = lax.axis_index("c")
  s = lax.axis_index("s")
  wid = c * NSUB + s

  _fill(ones, 128, 16, 1.0)
  _fill(zdeg, 128, 16, 0.0)

  def zacc(r, _):
    pltpu.sync_copy(rows_a, acc_sh.at[pl.ds(s * 1024 + r * 128, 128)])
    return 0

  def zdg(r, _):
    pltpu.sync_copy(zdeg, deg_sh.at[pl.ds(s * 1024 + r * 128, 128)])
    return 0

  pltpu.sync_copy(src2d.at[pl.ds(wid * 64, 64)], idxbuf)
  pltpu.sync_copy(dst2d.at[pl.ds(wid * 64, 64)], dstbuf)

  def scale4(j, _):
    for k in range(8):
      idxbuf[j, pl.ds(k * 16, 16)] = idxbuf[j, pl.ds(k * 16, 16)] * 4
    return 0

  lax.fori_loop(0, 64, scale4, 0)

  def plus1(j, _):
    for k in range(8):
      idxbuf[j, pl.ds(k * 16, 16)] = idxbuf[j, pl.ds(k * 16, 16)] + 1
    return 0

  for cc in range(NCHUNK):
    if cc > 0:
      lax.fori_loop(0, 64, plus1, 0)
    _fill(rows_a, 128, DC, 0.0)
    lax.fori_loop(0, 8, zacc, 0)
    if cc == 0:
      lax.fori_loop(0, 8, zdg, 0)
    plsc.subcore_barrier()

    pltpu.async_copy(table4.at[idxbuf.at[0]], rows_a, sem_a)

    def pair(g, _):
      j0 = 2 * g
      j1 = j0 + 1
      pltpu.async_copy(table4.at[idxbuf.at[j1]], rows_b, sem_b)
      pltpu.make_async_copy(table4.at[idxbuf.at[j0]], rows_a, sem_a).wait()
      pltpu.sync_copy(rows_a, acc_sh.at[dstbuf.at[j0]], add=True)
      if cc == 0:
        pltpu.sync_copy(ones, deg_sh.at[dstbuf.at[j0]], add=True)

      @pl.when(j1 < 63)
      def _():
        pltpu.async_copy(table4.at[idxbuf.at[j1 + 1]], rows_a, sem_a)

      pltpu.make_async_copy(table4.at[idxbuf.at[j1]], rows_b, sem_b).wait()
      pltpu.sync_copy(rows_b, acc_sh.at[dstbuf.at[j1]], add=True)
      if cc == 0:
        pltpu.sync_copy(ones, deg_sh.at[dstbuf.at[j1]], add=True)
      return 0

    lax.fori_loop(0, 32, pair, 0)

    plsc.subcore_barrier()
    pltpu.sync_copy(acc_sh.at[pl.ds(s * 1024, 1024)],
                    agg_part.at[c, cc, pl.ds(s * 1024, 1024)])
    if cc == 0:
      pltpu.sync_copy(deg_sh.at[pl.ds(s * 1024, 1024)],
                      deg_part.at[c, pl.ds(s * 1024, 1024)])


_sc_l0 = pl.kernel(
    _sc_l0_body,
    out_type=(
        jax.ShapeDtypeStruct((NCORE, NCHUNK, N1, DC), F32),
        jax.ShapeDtypeStruct((NCORE, N1, 16), F32),
    ),
    mesh=plsc.VectorSubcoreMesh(core_axis_name="c", subcore_axis_name="s"),
    compiler_params=pltpu.CompilerParams(use_tc_tiling_on_sc=False),
    scratch_types=[
        pltpu.VMEM_SHARED((N1, DC), F32),
        pltpu.VMEM_SHARED((N1, 16), F32),
        pltpu.VMEM((64, 128), jnp.int32),
        pltpu.VMEM((64, 128), jnp.int32),
        pltpu.VMEM((128, DC), F32),
        pltpu.VMEM((128, DC), F32),
        pltpu.VMEM((128, 16), F32),
        pltpu.VMEM((128, 16), F32),
        pltpu.SemaphoreType.DMA,
        pltpu.SemaphoreType.DMA,
    ],
)


def _sc_l1_body(h1, src2d, dst2d, agg_part, deg_part,
                acc_sh, deg_sh, srcbuf, dstbuf, rows_a, rows_b, ones, zdeg,
                sem_a, sem_b):
  c = lax.axis_index("c")
  s = lax.axis_index("s")
  wid = c * NSUB + s

  _fill(rows_a, 64, D, 0.0)
  _fill(zdeg, 64, 16, 0.0)
  _fill(ones, 64, 16, 1.0)

  def zacc(r, _):
    pltpu.sync_copy(rows_a, acc_sh.at[pl.ds(s * 256 + r * 64, 64)])
    return 0

  def zdg(r, _):
    pltpu.sync_copy(zdeg, deg_sh.at[pl.ds(s * 256 + r * 64, 64)])
    return 0

  lax.fori_loop(0, 4, zacc, 0)
  lax.fori_loop(0, 4, zdg, 0)

  pltpu.sync_copy(src2d.at[pl.ds(wid * 32, 32)], srcbuf)
  pltpu.sync_copy(dst2d.at[pl.ds(wid * 32, 32)], dstbuf)
  plsc.subcore_barrier()

  pltpu.async_copy(h1.at[srcbuf.at[0]], rows_a, sem_a)

  def pair(g, _):
    j0 = 2 * g
    j1 = j0 + 1
    pltpu.async_copy(h1.at[srcbuf.at[j1]], rows_b, sem_b)
    pltpu.make_async_copy(h1.at[srcbuf.at[j0]], rows_a, sem_a).wait()
    pltpu.sync_copy(rows_a, acc_sh.at[dstbuf.at[j0]], add=True)
    pltpu.sync_copy(ones, deg_sh.at[dstbuf.at[j0]], add=True)

    @pl.when(j1 < 31)
    def _():
      pltpu.async_copy(h1.at[srcbuf.at[j1 + 1]], rows_a, sem_a)

    pltpu.make_async_copy(h1.at[srcbuf.at[j1]], rows_b, sem_b).wait()
    pltpu.sync_copy(rows_b, acc_sh.at[dstbuf.at[j1]], add=True)
    pltpu.sync_copy(ones, deg_sh.at[dstbuf.at[j1]], add=True)
    return 0

  lax.fori_loop(0, 16, pair, 0)

  plsc.subcore_barrier()
  pltpu.sync_copy(acc_sh.at[pl.ds(s * 256, 256)],
                  agg_part.at[c, pl.ds(s * 256, 256)])
  pltpu.sync_copy(deg_sh.at[pl.ds(s * 256, 256)],
                  deg_part.at[c, pl.ds(s * 256, 256)])


_sc_l1 = pl.kernel(
    _sc_l1_body,
    out_type=(
        jax.ShapeDtypeStruct((NCORE, N2, D), F32),
        jax.ShapeDtypeStruct((NCORE, N2, 16), F32),
    ),
    mesh=plsc.VectorSubcoreMesh(core_axis_name="c", subcore_axis_name="s"),
    compiler_params=pltpu.CompilerParams(use_tc_tiling_on_sc=False),
    scratch_types=[
        pltpu.VMEM_SHARED((N2, D), F32),
        pltpu.VMEM_SHARED((N2, 16), F32),
        pltpu.VMEM((32, 64), jnp.int32),
        pltpu.VMEM((32, 64), jnp.int32),
        pltpu.VMEM((64, D), F32),
        pltpu.VMEM((64, D), F32),
        pltpu.VMEM((64, 16), F32),
        pltpu.VMEM((64, 16), F32),
        pltpu.SemaphoreType.DMA,
        pltpu.SemaphoreType.DMA,
    ],
)


def _dense0_body(x_ref, ap_ref, dg_ref, ws_ref, wn_ref, b_ref, o_ref):
  ap = ap_ref[...]
  agg = ap[0] + ap[1]
  agg = jnp.concatenate([agg[0], agg[1], agg[2], agg[3]], axis=-1)
  deg = jnp.max(dg_ref[0] + dg_ref[1], axis=1)
  neigh = agg / jnp.clip(deg, 1.0, None)[:, None]
  h = (jnp.dot(x_ref[...], ws_ref[...], preferred_element_type=F32)
       + jnp.dot(neigh, wn_ref[...], preferred_element_type=F32)
       + b_ref[...])
  o_ref[...] = jnp.maximum(h, 0.0)


_dense0 = pl.pallas_call(
    _dense0_body,
    grid=(N1 // 512,),
    in_specs=[
        pl.BlockSpec((512, D), lambda i: (i, 0)),
        pl.BlockSpec((NCORE, NCHUNK, 512, DC), lambda i: (0, 0, i, 0)),
        pl.BlockSpec((NCORE, 512, 16), lambda i: (0, i, 0)),
        pl.BlockSpec((D, D), lambda i: (0, 0)),
        pl.BlockSpec((D, D), lambda i: (0, 0)),
        pl.BlockSpec((1, D), lambda i: (0, 0)),
    ],
    out_specs=pl.BlockSpec((512, D), lambda i: (i, 0)),
    out_shape=jax.ShapeDtypeStruct((N1, D), F32),
)


def _head_body(h_ref, ap_ref, dg_ref, ws_ref, wn_ref, b_ref, wp_ref, wpp_ref,
               o_ref):
  ap = ap_ref[...]
  deg = jnp.max(dg_ref[0] + dg_ref[1], axis=1)
  neigh = (ap[0] + ap[1]) / jnp.clip(deg, 1.0, None)[:, None]
  hd = h_ref[...]
  h2 = (jnp.dot(hd, ws_ref[...], preferred_element_type=F32)
        + jnp.dot(neigh, wn_ref[...], preferred_element_type=F32)
        + b_ref[...])
  h2 = jnp.maximum(h2, 0.0)
  hcat = jnp.concatenate([h2, jnp.maximum(hd, 0.0)], axis=1)
  logits = lax.dot_general(hcat, wp_ref[...], (((1,), (1,)), ((), ())))
  mx = jnp.max(logits, axis=1, keepdims=True)
  eq = (logits >= mx).astype(F32)
  tri = (lax.broadcasted_iota(jnp.int32, (8, 8), 0)
         <= lax.broadcasted_iota(jnp.int32, (8, 8), 1)).astype(F32)
  cs = jnp.dot(eq, tri)
  onehot = eq * (cs == 1.0).astype(F32)
  allout = lax.dot_general(hcat, wpp_ref[...], (((1,), (1,)), ((), ())))
  ccg = lax.broadcasted_iota(jnp.int32, (8, 128), 0)
  jjg = lax.broadcasted_iota(jnp.int32, (8, 128), 1)
  expand = (jjg // 16 == ccg).astype(F32)
  maskfull = jnp.dot(onehot, expand)
  jk = lax.broadcasted_iota(jnp.int32, (128, 16), 0)
  kk = lax.broadcasted_iota(jnp.int32, (128, 16), 1)
  fold = (jk % 16 == kk).astype(F32)
  o_ref[...] = jnp.dot(allout * maskfull, fold)


_head = pl.pallas_call(
    _head_body,
    grid=(N2 // 512,),
    in_specs=[
        pl.BlockSpec((512, D), lambda i: (i, 0)),
        pl.BlockSpec((NCORE, 512, D), lambda i: (0, i, 0)),
        pl.BlockSpec((NCORE, 512, 16), lambda i: (0, i, 0)),
        pl.BlockSpec((D, D), lambda i: (0, 0)),
        pl.BlockSpec((D, D), lambda i: (0, 0)),
        pl.BlockSpec((1, D), lambda i: (0, 0)),
        pl.BlockSpec((8, 2 * D), lambda i: (0, 0)),
        pl.BlockSpec((128, 2 * D), lambda i: (0, 0)),
    ],
    out_specs=pl.BlockSpec((512, 16), lambda i: (i, 0)),
    out_shape=jax.ShapeDtypeStruct((N2, 16), F32),
)


def kernel(inputs, src0, dst0, src1, dst1, W_self0, W_neigh0, b0, W_self1,
           W_neigh1, b1, W_prompt, W_pp):
  table4 = inputs.reshape(N0 * NCHUNK, DC)
  agg0, deg0 = _sc_l0(table4, src0.reshape(E0 // 128, 128),
                      dst0.reshape(E0 // 128, 128))
  h1 = _dense0(inputs[:N1], agg0, deg0, W_self0, W_neigh0, b0.reshape(1, D))
  agg1, deg1 = _sc_l1(h1, src1.reshape(E1 // 64, 64),
                      dst1.reshape(E1 // 64, 64))
  out = _head(h1, agg1, deg1, W_self1, W_neigh1, b1.reshape(1, D), W_prompt,
              W_pp.reshape(8 * 16, 2 * D))
  return out
